# baseline (speedup 1.0000x reference)
import functools

import jax
import jax.numpy as jnp
from jax.experimental import pallas as pl
from jax.experimental.pallas import tpu as pltpu

N = 100000
D = 128
BN = 5000


def _fused_gate_kernel(x1_ref, x2_ref, w1a_ref, w1b_ref, b1_ref,
                       w2a_ref, w2b_ref, b2_ref, out_ref):
    x1 = x1_ref[...]
    x2 = x2_ref[...]
    logit1 = (jnp.dot(x1, w1a_ref[...], preferred_element_type=jnp.float32)
              + jnp.dot(x2, w1b_ref[...], preferred_element_type=jnp.float32)
              + b1_ref[...])
    logit2 = (jnp.dot(x1, w2a_ref[...], preferred_element_type=jnp.float32)
              + jnp.dot(x2, w2b_ref[...], preferred_element_type=jnp.float32)
              + b2_ref[...])
    g1 = 0.5 * jnp.tanh(0.5 * logit1) + 0.5
    g2 = 0.5 * jnp.tanh(0.5 * logit2) + 0.5
    out_ref[...] = g1 * x1 + g2 * x2


@jax.jit
def _fused_gate(h_coa, h_aoa, W1, b1, W2, b2):
    n = h_coa.shape[0]
    grid = (n // BN,)
    row_block = pl.BlockSpec((BN, D), lambda i: (i, 0))
    full = pl.BlockSpec((D, D), lambda i: (0, 0))
    bias = pl.BlockSpec((1, D), lambda i: (0, 0))
    return pl.pallas_call(
        _fused_gate_kernel,
        grid=grid,
        in_specs=[row_block, row_block, full, full, bias, full, full, bias],
        out_specs=row_block,
        out_shape=jax.ShapeDtypeStruct((n, D), jnp.float32),
        compiler_params=pltpu.CompilerParams(
            dimension_semantics=("parallel",)),
    )(h_coa, h_aoa, W1[:D], W1[D:], b1.reshape(1, D), W2[:D], W2[D:],
      b2.reshape(1, D))


def kernel(h_coa, h_aoa, W1, b1, W2, b2):
    return _fused_gate(h_coa, h_aoa, W1, b1, W2, b2)


# speedup vs baseline: 1.2149x; 1.2149x over previous
import functools

import jax
import jax.numpy as jnp
from jax.experimental import pallas as pl
from jax.experimental.pallas import tpu as pltpu

N = 100000
D = 128
BN = 10000


def _fused_gate_kernel(x1_ref, x2_ref, w1a_ref, w1b_ref, b1_ref,
                       w2a_ref, w2b_ref, b2_ref, out_ref):
    x1 = x1_ref[...]
    x2 = x2_ref[...]
    logit1 = (jnp.dot(x1, w1a_ref[...], preferred_element_type=jnp.float32)
              + jnp.dot(x2, w1b_ref[...], preferred_element_type=jnp.float32)
              + b1_ref[...])
    logit2 = (jnp.dot(x1, w2a_ref[...], preferred_element_type=jnp.float32)
              + jnp.dot(x2, w2b_ref[...], preferred_element_type=jnp.float32)
              + b2_ref[...])
    g1 = 0.5 * jnp.tanh(0.5 * logit1) + 0.5
    g2 = 0.5 * jnp.tanh(0.5 * logit2) + 0.5
    out_ref[...] = g1 * x1 + g2 * x2


@jax.jit
def _fused_gate(h_coa, h_aoa, W1, b1, W2, b2):
    n = h_coa.shape[0]
    grid = (n // BN,)
    row_block = pl.BlockSpec((BN, D), lambda i: (i, 0))
    full = pl.BlockSpec((D, D), lambda i: (0, 0))
    bias = pl.BlockSpec((1, D), lambda i: (0, 0))
    return pl.pallas_call(
        _fused_gate_kernel,
        grid=grid,
        in_specs=[row_block, row_block, full, full, bias, full, full, bias],
        out_specs=row_block,
        out_shape=jax.ShapeDtypeStruct((n, D), jnp.float32),
        compiler_params=pltpu.CompilerParams(
            dimension_semantics=("parallel",)),
    )(h_coa, h_aoa, W1[:D], W1[D:], b1.reshape(1, D), W2[:D], W2[D:],
      b2.reshape(1, D))


def kernel(h_coa, h_aoa, W1, b1, W2, b2):
    return _fused_gate(h_coa, h_aoa, W1, b1, W2, b2)


# speedup vs baseline: 1.2171x; 1.0018x over previous
import functools

import jax
import jax.numpy as jnp
from jax.experimental import pallas as pl
from jax.experimental.pallas import tpu as pltpu

N = 100000
D = 128
BN = 10000


def _fused_gate_kernel(x1_ref, x2_ref, w1a_ref, w1b_ref, b1_ref,
                       w2a_ref, w2b_ref, b2_ref, out_ref):
    x1 = x1_ref[...]
    x2 = x2_ref[...]
    logit1 = (jnp.dot(x1, w1a_ref[...], preferred_element_type=jnp.float32)
              + jnp.dot(x2, w1b_ref[...], preferred_element_type=jnp.float32)
              + b1_ref[...])
    logit2 = (jnp.dot(x1, w2a_ref[...], preferred_element_type=jnp.float32)
              + jnp.dot(x2, w2b_ref[...], preferred_element_type=jnp.float32)
              + b2_ref[...])
    g1 = 0.5 * jnp.tanh(0.5 * logit1) + 0.5
    g2 = 0.5 * jnp.tanh(0.5 * logit2) + 0.5
    out_ref[...] = g1 * x1 + g2 * x2


@jax.jit
def _fused_gate(h_coa, h_aoa, W1, b1, W2, b2):
    n = h_coa.shape[0]
    grid = (n // BN,)
    row_block = pl.BlockSpec((BN, D), lambda i: (i, 0))
    full = pl.BlockSpec((D, D), lambda i: (0, 0))
    bias = pl.BlockSpec((1, D), lambda i: (0, 0))
    return pl.pallas_call(
        _fused_gate_kernel,
        grid=grid,
        in_specs=[row_block, row_block, full, full, bias, full, full, bias],
        out_specs=row_block,
        out_shape=jax.ShapeDtypeStruct((n, D), jnp.float32),
        compiler_params=pltpu.CompilerParams(
            dimension_semantics=("arbitrary",)),
    )(h_coa, h_aoa, W1[:D], W1[D:], b1.reshape(1, D), W2[:D], W2[D:],
      b2.reshape(1, D))


def kernel(h_coa, h_aoa, W1, b1, W2, b2):
    return _fused_gate(h_coa, h_aoa, W1, b1, W2, b2)
